# baseline (speedup 1.0000x reference)
import functools

import jax
import jax.numpy as jnp
from jax import lax
from jax.experimental import pallas as pl
from jax.experimental.pallas import tpu as pltpu
from jax.experimental.pallas import tpu_sc as plsc

B, L, V, D = 1024, 200, 1000000, 64
NROWS = B * L
NW = 32
RW = NROWS // NW
CHUNK = 640
SUB = 128
NSUB = CHUNK // SUB
NCHUNK = RW // CHUNK


def _sc_gather_pairs(idx2, table2):
    mesh = plsc.VectorSubcoreMesh(core_axis_name="c", subcore_axis_name="s")

    @functools.partial(
        pl.kernel,
        out_type=jax.ShapeDtypeStruct((NROWS, 128), jnp.float32),
        mesh=mesh,
        scratch_types=[
            pltpu.VMEM((CHUNK,), jnp.int32),
            pltpu.VMEM((CHUNK, 128), jnp.float32),
            pltpu.SemaphoreType.DMA,
        ],
    )
    def gather_kernel(idx_hbm, table_hbm, out_hbm, idx_v, rows_v, sem):
        wid = lax.axis_index("s") * 2 + lax.axis_index("c")
        base = wid * RW

        def chunk_body(g, carry):
            off = base + g * CHUNK
            pltpu.sync_copy(idx_hbm.at[pl.ds(off, CHUNK)], idx_v)
            cps = []
            for j in range(NSUB):
                cps.append(
                    pltpu.async_copy(
                        table_hbm.at[idx_v.at[pl.ds(j * SUB, SUB)]],
                        rows_v.at[pl.ds(j * SUB, SUB)],
                        sem,
                    )
                )
            for cp in cps:
                cp.wait()
            pltpu.sync_copy(rows_v, out_hbm.at[pl.ds(off, CHUNK)])
            return carry

        lax.fori_loop(0, NCHUNK, chunk_body, 0)

    return gather_kernel(idx2, table2)


BB1 = 16
RB1 = BB1 * L


def _tc_out1(g128, par, ph2, pos_table, unit_embed):
    def body(g_ref, par_ref, ph_ref, pos_ref, ue_ref, out_ref):
        g = g_ref[...]
        left = g[:, :D]
        right = g[:, D:]
        pm = par_ref[...]
        half = left + pm * (right - left)
        unit = lax.dot_general(
            ph_ref[...], ue_ref[...],
            (((1,), (0,)), ((), ())),
            preferred_element_type=jnp.float32,
        )
        posb = jnp.broadcast_to(pos_ref[...][None], (BB1, L, D)).reshape(RB1, D)
        out_ref[...] = half + unit + posb

    return pl.pallas_call(
        body,
        grid=(B // BB1,),
        in_specs=[
            pl.BlockSpec((RB1, 128), lambda i: (i, 0)),
            pl.BlockSpec((RB1, 1), lambda i: (i, 0)),
            pl.BlockSpec((RB1, 7), lambda i: (i, 0)),
            pl.BlockSpec((L, D), lambda i: (0, 0)),
            pl.BlockSpec((7, D), lambda i: (0, 0)),
        ],
        out_specs=pl.BlockSpec((RB1, D), lambda i: (i, 0)),
        out_shape=jax.ShapeDtypeStruct((NROWS, D), jnp.float32),
    )(g128, par, ph2, pos_table, unit_embed)


BB2 = 64


def _tc_out2(meta_info, padding, case_embed):
    def body(meta_ref, pad_ref, case_ref, out_ref):
        m = meta_ref[...]
        p = pad_ref[...]
        prod = m[:, None, :] * p[None, :, :]
        res = lax.dot_general(
            prod.reshape(BB2 * L, D), case_ref[...],
            (((1,), (0,)), ((), ())),
            preferred_element_type=jnp.float32,
        )
        out_ref[...] = res.reshape(BB2, L, D)

    return pl.pallas_call(
        body,
        grid=(B // BB2,),
        in_specs=[
            pl.BlockSpec((BB2, D), lambda i: (i, 0)),
            pl.BlockSpec((L, D), lambda i: (0, 0)),
            pl.BlockSpec((D, D), lambda i: (0, 0)),
        ],
        out_specs=pl.BlockSpec((BB2, L, D), lambda i: (i, 0, 0)),
        out_shape=jax.ShapeDtypeStruct((B, L, D), jnp.float32),
    )(meta_info, padding, case_embed)


def kernel(sequence, meta_info, ph_dimensions, token_table, pos_table,
           case_embed, unit_embed, padding):
    seq_flat = sequence.reshape(NROWS).astype(jnp.int32)
    table2 = token_table.reshape(V // 2, 128)
    idx2 = seq_flat // 2
    par = (seq_flat % 2).astype(jnp.float32).reshape(NROWS, 1)
    g128 = _sc_gather_pairs(idx2, table2)
    ph2 = ph_dimensions.astype(jnp.float32).reshape(NROWS, 7)
    out1 = _tc_out1(g128, par, ph2, pos_table, unit_embed).reshape(B, L, D)
    out2 = _tc_out2(meta_info, padding, case_embed)
    return (out1, out2)


# speedup vs baseline: 1.7896x; 1.7896x over previous
import functools

import jax
import jax.numpy as jnp
from jax import lax
from jax.experimental import pallas as pl
from jax.experimental.pallas import tpu as pltpu
from jax.experimental.pallas import tpu_sc as plsc

B, L, V, D = 1024, 200, 1000000, 64
NROWS = B * L
NW = 32
RW = NROWS // NW
CHUNK = 640
SUB = 128
NSUB = CHUNK // SUB
NCHUNK = RW // CHUNK

CW = 8192
NPAIR = CW // 2


def _tc_repack(table_t):
    def body(x_ref, o_ref):
        t = jnp.transpose(x_ref[...], (1, 0))
        r = t.reshape(NPAIR, 2, D)
        o_ref[...] = jnp.concatenate([r[:, 0, :], r[:, 1, :]], axis=1)

    return pl.pallas_call(
        body,
        grid=((V + CW - 1) // CW,),
        in_specs=[pl.BlockSpec((D, CW), lambda i: (0, i))],
        out_specs=pl.BlockSpec((NPAIR, 128), lambda i: (i, 0)),
        out_shape=jax.ShapeDtypeStruct((V // 2, 128), jnp.float32),
    )(table_t)


def _sc_gather_pairs(idx2, table2):
    mesh = plsc.VectorSubcoreMesh(core_axis_name="c", subcore_axis_name="s")

    @functools.partial(
        pl.kernel,
        out_type=jax.ShapeDtypeStruct((NROWS, 128), jnp.float32),
        mesh=mesh,
        scratch_types=[
            pltpu.VMEM((CHUNK,), jnp.int32),
            pltpu.VMEM((CHUNK, 128), jnp.float32),
            pltpu.SemaphoreType.DMA,
        ],
    )
    def gather_kernel(idx_hbm, table_hbm, out_hbm, idx_v, rows_v, sem):
        wid = lax.axis_index("s") * 2 + lax.axis_index("c")
        base = wid * RW

        def chunk_body(g, carry):
            off = base + g * CHUNK
            pltpu.sync_copy(idx_hbm.at[pl.ds(off, CHUNK)], idx_v)
            cps = []
            for j in range(NSUB):
                cps.append(
                    pltpu.async_copy(
                        table_hbm.at[idx_v.at[pl.ds(j * SUB, SUB)]],
                        rows_v.at[pl.ds(j * SUB, SUB)],
                        sem,
                    )
                )
            for cp in cps:
                cp.wait()
            pltpu.sync_copy(rows_v, out_hbm.at[pl.ds(off, CHUNK)])
            return carry

        lax.fori_loop(0, NCHUNK, chunk_body, 0)

    return gather_kernel(idx2, table2)


LB = 8


def _tc_out1t(g128, par3, ph_t, pos_t, unit_embed):
    def body(g_ref, par_ref, ph_ref, pos_ref, ue_ref, out_ref):
        g3 = g_ref[...].reshape(LB, B, 128)
        pm = par_ref[...]
        ue = ue_ref[...]
        for j in range(LB):
            gl = jnp.transpose(g3[j, :, :D], (1, 0))
            gr = jnp.transpose(g3[j, :, D:], (1, 0))
            half = gl + pm[j] * (gr - gl)
            unit = lax.dot_general(
                ue, ph_ref[:, j, :],
                (((0,), (0,)), ((), ())),
                preferred_element_type=jnp.float32,
            )
            out_ref[j] = half + unit + pos_ref[0, :, j][:, None]

    return pl.pallas_call(
        body,
        grid=(L // LB,),
        in_specs=[
            pl.BlockSpec((LB * B, 128), lambda i: (i, 0)),
            pl.BlockSpec((LB, 1, B), lambda i: (i, 0, 0)),
            pl.BlockSpec((7, LB, B), lambda i: (0, i, 0)),
            pl.BlockSpec((1, D, LB), lambda i: (i, 0, 0)),
            pl.BlockSpec((7, D), lambda i: (0, 0)),
        ],
        out_specs=pl.BlockSpec((LB, D, B), lambda i: (i, 0, 0)),
        out_shape=jax.ShapeDtypeStruct((L, D, B), jnp.float32),
    )(g128, par3, ph_t, pos_t, unit_embed)


LB2 = 8


def _tc_out2t(meta_t, pad_t, case_embed):
    def body(meta_ref, pad_ref, case_ref, out_ref):
        mt = meta_ref[...]
        case = case_ref[...]
        for j in range(LB2):
            rhs = mt * pad_ref[0, :, j][:, None]
            out_ref[j] = lax.dot_general(
                case, rhs,
                (((0,), (0,)), ((), ())),
                preferred_element_type=jnp.float32,
            )

    return pl.pallas_call(
        body,
        grid=(L // LB2,),
        in_specs=[
            pl.BlockSpec((D, B), lambda i: (0, 0)),
            pl.BlockSpec((1, D, LB2), lambda i: (i, 0, 0)),
            pl.BlockSpec((D, D), lambda i: (0, 0)),
        ],
        out_specs=pl.BlockSpec((LB2, D, B), lambda i: (i, 0, 0)),
        out_shape=jax.ShapeDtypeStruct((L, D, B), jnp.float32),
    )(meta_t, pad_t, case_embed)


def kernel(sequence, meta_info, ph_dimensions, token_table, pos_table,
           case_embed, unit_embed, padding):
    table_t = jnp.transpose(token_table, (1, 0))
    table2 = _tc_repack(table_t)

    seq_t = jnp.transpose(sequence, (1, 0))
    idx_t = seq_t.reshape(NROWS).astype(jnp.int32)
    idx2 = idx_t // 2
    par3 = (idx_t % 2).astype(jnp.float32).reshape(L, 1, B)

    g128 = _sc_gather_pairs(idx2, table2)

    ph_t = jnp.transpose(ph_dimensions.astype(jnp.float32), (2, 1, 0))
    pos_t = jnp.transpose(pos_table, (1, 0))
    pos_r = jnp.transpose(pos_t.reshape(D, L // LB, LB), (1, 0, 2))
    out1t = _tc_out1t(g128, par3, ph_t, pos_r, unit_embed)

    meta_t = jnp.transpose(meta_info, (1, 0))
    pad_t = jnp.transpose(padding, (1, 0))
    pad_r = jnp.transpose(pad_t.reshape(D, L // LB2, LB2), (1, 0, 2))
    out2t = _tc_out2t(meta_t, pad_r, case_embed)

    out1 = jnp.transpose(out1t, (2, 0, 1))
    out2 = jnp.transpose(out2t, (2, 0, 1))
    return (out1, out2)


# speedup vs baseline: 2.3953x; 1.3385x over previous
import functools

import jax
import jax.numpy as jnp
from jax import lax
from jax.experimental import pallas as pl
from jax.experimental.pallas import tpu as pltpu
from jax.experimental.pallas import tpu_sc as plsc

B, L, V, D = 1024, 200, 1000000, 64
NROWS = B * L
NW = 32
RW = NROWS // NW
CHUNK = 640
SUB = 128
NSUB = CHUNK // SUB
NCHUNK = RW // CHUNK

H = 524288
NPAIR = 4096
NREP = H // NPAIR


def _tc_repack(table_t, eye64):
    def body(xl_ref, xr_ref, eye_ref, o_ref):
        e = eye_ref[...]
        tl = lax.dot_general(
            xl_ref[...], e, (((0,), (0,)), ((), ())),
            preferred_element_type=jnp.float32)
        tr = lax.dot_general(
            xr_ref[...], e, (((0,), (0,)), ((), ())),
            preferred_element_type=jnp.float32)
        o_ref[:, :D] = tl
        o_ref[:, D:] = tr

    return pl.pallas_call(
        body,
        grid=(NREP,),
        in_specs=[
            pl.BlockSpec((D, NPAIR), lambda i: (0, i)),
            pl.BlockSpec((D, NPAIR), lambda i: (0, jnp.minimum(i + NREP, V // NPAIR))),
            pl.BlockSpec((D, D), lambda i: (0, 0)),
        ],
        out_specs=pl.BlockSpec((NPAIR, 128), lambda i: (i, 0)),
        out_shape=jax.ShapeDtypeStruct((H, 128), jnp.float32),
    )(table_t, table_t, eye64)


def _sc_gather_pairs(idx2, table2):
    mesh = plsc.VectorSubcoreMesh(core_axis_name="c", subcore_axis_name="s")

    @functools.partial(
        pl.kernel,
        out_type=jax.ShapeDtypeStruct((NROWS, 128), jnp.float32),
        mesh=mesh,
        scratch_types=[
            pltpu.VMEM((CHUNK,), jnp.int32),
            pltpu.VMEM((CHUNK, 128), jnp.float32),
            pltpu.SemaphoreType.DMA,
        ],
    )
    def gather_kernel(idx_hbm, table_hbm, out_hbm, idx_v, rows_v, sem):
        wid = lax.axis_index("s") * 2 + lax.axis_index("c")
        base = wid * RW

        def chunk_body(g, carry):
            off = base + g * CHUNK
            pltpu.sync_copy(idx_hbm.at[pl.ds(off, CHUNK)], idx_v)
            cps = []
            for j in range(NSUB):
                cps.append(
                    pltpu.async_copy(
                        table_hbm.at[idx_v.at[pl.ds(j * SUB, SUB)]],
                        rows_v.at[pl.ds(j * SUB, SUB)],
                        sem,
                    )
                )
            for cp in cps:
                cp.wait()
            pltpu.sync_copy(rows_v, out_hbm.at[pl.ds(off, CHUNK)])
            return carry

        lax.fori_loop(0, NCHUNK, chunk_body, 0)

    return gather_kernel(idx2, table2)


LB = 8


def _tc_out1t(g128, par3, ph_t, pos_t, unit_embed, eyelr):
    def body(g_ref, par_ref, ph_ref, pos_ref, ue_ref, eye_ref, out_ref):
        g3 = g_ref[...].reshape(LB, B, 128)
        pm = par_ref[...]
        ue = ue_ref[...]
        el = eye_ref[0]
        er = eye_ref[1]
        for j in range(LB):
            gl = lax.dot_general(
                el, g3[j], (((1,), (1,)), ((), ())),
                preferred_element_type=jnp.float32)
            gr = lax.dot_general(
                er, g3[j], (((1,), (1,)), ((), ())),
                preferred_element_type=jnp.float32)
            half = gl + pm[j] * (gr - gl)
            unit = lax.dot_general(
                ue, ph_ref[:, j, :],
                (((0,), (0,)), ((), ())),
                preferred_element_type=jnp.float32,
            )
            out_ref[j] = half + unit + pos_ref[0, :, j][:, None]

    return pl.pallas_call(
        body,
        grid=(L // LB,),
        in_specs=[
            pl.BlockSpec((LB * B, 128), lambda i: (i, 0)),
            pl.BlockSpec((LB, 1, B), lambda i: (i, 0, 0)),
            pl.BlockSpec((7, LB, B), lambda i: (0, i, 0)),
            pl.BlockSpec((1, D, LB), lambda i: (i, 0, 0)),
            pl.BlockSpec((7, D), lambda i: (0, 0)),
            pl.BlockSpec((2, D, 128), lambda i: (0, 0, 0)),
        ],
        out_specs=pl.BlockSpec((LB, D, B), lambda i: (i, 0, 0)),
        out_shape=jax.ShapeDtypeStruct((L, D, B), jnp.float32),
    )(g128, par3, ph_t, pos_t, unit_embed, eyelr)


LB2 = 8


def _tc_out2t(meta_t, pad_t, case_embed):
    def body(meta_ref, pad_ref, case_ref, out_ref):
        mt = meta_ref[...]
        case = case_ref[...]
        for j in range(LB2):
            rhs = mt * pad_ref[0, :, j][:, None]
            out_ref[j] = lax.dot_general(
                case, rhs,
                (((0,), (0,)), ((), ())),
                preferred_element_type=jnp.float32,
            )

    return pl.pallas_call(
        body,
        grid=(L // LB2,),
        in_specs=[
            pl.BlockSpec((D, B), lambda i: (0, 0)),
            pl.BlockSpec((1, D, LB2), lambda i: (i, 0, 0)),
            pl.BlockSpec((D, D), lambda i: (0, 0)),
        ],
        out_specs=pl.BlockSpec((LB2, D, B), lambda i: (i, 0, 0)),
        out_shape=jax.ShapeDtypeStruct((L, D, B), jnp.float32),
    )(meta_t, pad_t, case_embed)


def kernel(sequence, meta_info, ph_dimensions, token_table, pos_table,
           case_embed, unit_embed, padding):
    table_t = jnp.transpose(token_table, (1, 0))
    eye64 = jnp.eye(D, dtype=jnp.float32)
    table2 = _tc_repack(table_t, eye64)

    seq_t = jnp.transpose(sequence, (1, 0))
    idx_t = seq_t.reshape(NROWS).astype(jnp.int32)
    high = idx_t >= H
    idx2 = jnp.where(high, idx_t - H, idx_t)
    par3 = high.astype(jnp.float32).reshape(L, 1, B)
    eyelr = jnp.stack([jnp.eye(D, 128, dtype=jnp.float32),
                       jnp.eye(D, 128, k=D, dtype=jnp.float32)])

    g128 = _sc_gather_pairs(idx2, table2)

    ph_t = jnp.transpose(ph_dimensions.astype(jnp.float32), (2, 1, 0))
    pos_t = jnp.transpose(pos_table, (1, 0))
    pos_r = jnp.transpose(pos_t.reshape(D, L // LB, LB), (1, 0, 2))
    out1t = _tc_out1t(g128, par3, ph_t, pos_r, unit_embed, eyelr)

    meta_t = jnp.transpose(meta_info, (1, 0))
    pad_t = jnp.transpose(padding, (1, 0))
    pad_r = jnp.transpose(pad_t.reshape(D, L // LB2, LB2), (1, 0, 2))
    out2t = _tc_out2t(meta_t, pad_r, case_embed)

    out1 = jnp.transpose(out1t, (2, 0, 1))
    out2 = jnp.transpose(out2t, (2, 0, 1))
    return (out1, out2)


# speedup vs baseline: 2.6100x; 1.0896x over previous
import functools

import jax
import jax.numpy as jnp
from jax import lax
from jax.experimental import pallas as pl
from jax.experimental.pallas import tpu as pltpu
from jax.experimental.pallas import tpu_sc as plsc

B, L, V, D = 1024, 200, 1000000, 64
NROWS = B * L
NW = 32
RW = NROWS // NW
CHUNK = 640
SUB = 128
NSUB = CHUNK // SUB
NCHUNK = RW // CHUNK

H = 524288
NPAIR = 8192
NREP = H // NPAIR


def _tc_repack(table_t, eye64):
    def body(xl_ref, xr_ref, eye_ref, o_ref):
        e = eye_ref[...]
        tl = lax.dot_general(
            xl_ref[...], e, (((0,), (0,)), ((), ())),
            preferred_element_type=jnp.float32)
        tr = lax.dot_general(
            xr_ref[...], e, (((0,), (0,)), ((), ())),
            preferred_element_type=jnp.float32)
        o_ref[:, :D] = tl
        o_ref[:, D:] = tr

    return pl.pallas_call(
        body,
        grid=(NREP,),
        in_specs=[
            pl.BlockSpec((D, NPAIR), lambda i: (0, i)),
            pl.BlockSpec((D, NPAIR), lambda i: (0, jnp.minimum(i + NREP, V // NPAIR))),
            pl.BlockSpec((D, D), lambda i: (0, 0)),
        ],
        out_specs=pl.BlockSpec((NPAIR, 128), lambda i: (i, 0)),
        out_shape=jax.ShapeDtypeStruct((H, 128), jnp.float32),
    )(table_t, table_t, eye64)


def _sc_gather_pairs(idx2, table2):
    mesh = plsc.VectorSubcoreMesh(core_axis_name="c", subcore_axis_name="s")

    @functools.partial(
        pl.kernel,
        out_type=jax.ShapeDtypeStruct((NROWS, 128), jnp.float32),
        mesh=mesh,
        scratch_types=[
            pltpu.VMEM((RW,), jnp.int32),
            pltpu.VMEM((CHUNK, 128), jnp.float32),
            pltpu.SemaphoreType.DMA,
        ],
    )
    def gather_kernel(idx_hbm, table_hbm, out_hbm, idx_v, rows_v, sem):
        wid = lax.axis_index("s") * 2 + lax.axis_index("c")
        base = wid * RW
        pltpu.sync_copy(idx_hbm.at[pl.ds(base, RW)], idx_v)

        def chunk_body(g, carry):
            off = g * CHUNK
            cps = []
            for j in range(NSUB):
                cps.append(
                    pltpu.async_copy(
                        table_hbm.at[idx_v.at[pl.ds(off + j * SUB, SUB)]],
                        rows_v.at[pl.ds(j * SUB, SUB)],
                        sem,
                    )
                )
            for cp in cps:
                cp.wait()
            pltpu.sync_copy(rows_v, out_hbm.at[pl.ds(base + off, CHUNK)])
            return carry

        lax.fori_loop(0, NCHUNK, chunk_body, 0)

    return gather_kernel(idx2, table2)


LB = 8


def _tc_out1t(g128, par3, ph_t, pos_t, unit_embed, eyelr):
    def body(g_ref, par_ref, ph_ref, pos_ref, ue_ref, eye_ref, out_ref):
        g3 = g_ref[...].reshape(LB, B, 128)
        pm = par_ref[...]
        ue = ue_ref[...]
        el = eye_ref[0]
        er = eye_ref[1]
        for j in range(LB):
            gl = lax.dot_general(
                el, g3[j], (((1,), (1,)), ((), ())),
                preferred_element_type=jnp.float32)
            gr = lax.dot_general(
                er, g3[j], (((1,), (1,)), ((), ())),
                preferred_element_type=jnp.float32)
            half = gl + pm[j] * (gr - gl)
            unit = lax.dot_general(
                ue, ph_ref[:, j, :],
                (((0,), (0,)), ((), ())),
                preferred_element_type=jnp.float32,
            )
            out_ref[j] = half + unit + pos_ref[0, :, j][:, None]

    return pl.pallas_call(
        body,
        grid=(L // LB,),
        in_specs=[
            pl.BlockSpec((LB * B, 128), lambda i: (i, 0)),
            pl.BlockSpec((LB, 1, B), lambda i: (i, 0, 0)),
            pl.BlockSpec((7, LB, B), lambda i: (0, i, 0)),
            pl.BlockSpec((1, D, LB), lambda i: (i, 0, 0)),
            pl.BlockSpec((7, D), lambda i: (0, 0)),
            pl.BlockSpec((2, D, 128), lambda i: (0, 0, 0)),
        ],
        out_specs=pl.BlockSpec((LB, D, B), lambda i: (i, 0, 0)),
        out_shape=jax.ShapeDtypeStruct((L, D, B), jnp.float32),
    )(g128, par3, ph_t, pos_t, unit_embed, eyelr)


LB2 = 8


def _tc_out2t(meta_t, pad_t, case_embed):
    def body(meta_ref, pad_ref, case_ref, out_ref):
        mt = meta_ref[...]
        case = case_ref[...]
        for j in range(LB2):
            rhs = mt * pad_ref[0, :, j][:, None]
            out_ref[j] = lax.dot_general(
                case, rhs,
                (((0,), (0,)), ((), ())),
                preferred_element_type=jnp.float32,
            )

    return pl.pallas_call(
        body,
        grid=(L // LB2,),
        in_specs=[
            pl.BlockSpec((D, B), lambda i: (0, 0)),
            pl.BlockSpec((1, D, LB2), lambda i: (i, 0, 0)),
            pl.BlockSpec((D, D), lambda i: (0, 0)),
        ],
        out_specs=pl.BlockSpec((LB2, D, B), lambda i: (i, 0, 0)),
        out_shape=jax.ShapeDtypeStruct((L, D, B), jnp.float32),
    )(meta_t, pad_t, case_embed)


def kernel(sequence, meta_info, ph_dimensions, token_table, pos_table,
           case_embed, unit_embed, padding):
    table_t = jnp.transpose(token_table, (1, 0))
    eye64 = jnp.eye(D, dtype=jnp.float32)
    table2 = _tc_repack(table_t, eye64)

    seq_t = jnp.transpose(sequence, (1, 0))
    idx_t = seq_t.reshape(NROWS).astype(jnp.int32)
    high = idx_t >= H
    idx2 = jnp.where(high, idx_t - H, idx_t)
    par3 = high.astype(jnp.float32).reshape(L, 1, B)
    eyelr = jnp.stack([jnp.eye(D, 128, dtype=jnp.float32),
                       jnp.eye(D, 128, k=D, dtype=jnp.float32)])

    g128 = _sc_gather_pairs(idx2, table2)

    ph_t = jnp.transpose(ph_dimensions.astype(jnp.float32), (2, 1, 0))
    pos_t = jnp.transpose(pos_table, (1, 0))
    pos_r = jnp.transpose(pos_t.reshape(D, L // LB, LB), (1, 0, 2))
    out1t = _tc_out1t(g128, par3, ph_t, pos_r, unit_embed, eyelr)

    meta_t = jnp.transpose(meta_info, (1, 0))
    pad_t = jnp.transpose(padding, (1, 0))
    pad_r = jnp.transpose(pad_t.reshape(D, L // LB2, LB2), (1, 0, 2))
    out2t = _tc_out2t(meta_t, pad_r, case_embed)

    out1 = jnp.transpose(out1t, (2, 0, 1))
    out2 = jnp.transpose(out2t, (2, 0, 1))
    return (out1, out2)


# speedup vs baseline: 2.6406x; 1.0117x over previous
import functools

import jax
import jax.numpy as jnp
from jax import lax
from jax.experimental import pallas as pl
from jax.experimental.pallas import tpu as pltpu
from jax.experimental.pallas import tpu_sc as plsc

B, L, V, D = 1024, 200, 1000000, 64
NROWS = B * L
NW = 32
RW = NROWS // NW
CHUNK = 320
SUB = 64
NSUB = CHUNK // SUB
NCHUNK = RW // CHUNK

H = 524288
NPAIR = 8192
NREP = H // NPAIR


def _tc_repack(table_t, eye64):
    def body(xl_ref, xr_ref, eye_ref, o_ref):
        e = eye_ref[...]
        tl = lax.dot_general(
            xl_ref[...], e, (((0,), (0,)), ((), ())),
            preferred_element_type=jnp.float32)
        tr = lax.dot_general(
            xr_ref[...], e, (((0,), (0,)), ((), ())),
            preferred_element_type=jnp.float32)
        o_ref[:, :D] = tl
        o_ref[:, D:] = tr

    return pl.pallas_call(
        body,
        grid=(NREP,),
        in_specs=[
            pl.BlockSpec((D, NPAIR), lambda i: (0, i)),
            pl.BlockSpec((D, NPAIR), lambda i: (0, jnp.minimum(i + NREP, V // NPAIR))),
            pl.BlockSpec((D, D), lambda i: (0, 0)),
        ],
        out_specs=pl.BlockSpec((NPAIR, 128), lambda i: (i, 0)),
        out_shape=jax.ShapeDtypeStruct((H, 128), jnp.float32),
    )(table_t, table_t, eye64)


def _sc_gather_pairs(idx2, table2):
    mesh = plsc.VectorSubcoreMesh(core_axis_name="c", subcore_axis_name="s")

    @functools.partial(
        pl.kernel,
        out_type=jax.ShapeDtypeStruct((NROWS, 128), jnp.float32),
        mesh=mesh,
        scratch_types=[
            pltpu.VMEM((RW,), jnp.int32),
            pltpu.VMEM((2, CHUNK, 128), jnp.float32),
            pltpu.SemaphoreType.DMA,
            pltpu.SemaphoreType.DMA,
            pltpu.SemaphoreType.DMA,
        ],
    )
    def gather_kernel(idx_hbm, table_hbm, out_hbm, idx_v, rows_v,
                      gsem0, gsem1, ssem):
        wid = lax.axis_index("s") * 2 + lax.axis_index("c")
        base = wid * RW
        pltpu.sync_copy(idx_hbm.at[pl.ds(base, RW)], idx_v)
        gsems = (gsem0, gsem1)

        def fire(slot, g, sem):
            cps = []
            for j in range(NSUB):
                cps.append(
                    pltpu.async_copy(
                        table_hbm.at[idx_v.at[pl.ds(g * CHUNK + j * SUB, SUB)]],
                        rows_v.at[slot].at[pl.ds(j * SUB, SUB)],
                        sem,
                    )
                )
            return cps

        def drain(cps):
            for cp in cps:
                cp.wait()

        cps0 = fire(0, 0, gsem0)

        def pair_body(h, carry):
            a = 2 * h
            b = a + 1
            for j in range(NSUB):
                pltpu.make_async_copy(
                    table_hbm.at[idx_v.at[pl.ds(a * CHUNK + j * SUB, SUB)]],
                    rows_v.at[0].at[pl.ds(j * SUB, SUB)],
                    gsem0,
                ).wait()
            cb = fire(1, b, gsem1)
            st_a = pltpu.async_copy(
                rows_v.at[0], out_hbm.at[pl.ds(base + a * CHUNK, CHUNK)], ssem)
            drain(cb)
            st_a.wait()

            @pl.when(h + 1 < NCHUNK // 2)
            def _():
                fire(0, a + 2, gsem0)

            st_b = pltpu.async_copy(
                rows_v.at[1], out_hbm.at[pl.ds(base + b * CHUNK, CHUNK)], ssem)
            st_b.wait()
            return carry

        lax.fori_loop(0, NCHUNK // 2, pair_body, 0)

    return gather_kernel(idx2, table2)


LB = 8


def _tc_out1t(g128, par3, ph_t, pos_t, unit_embed, eyelr):
    def body(g_ref, par_ref, ph_ref, pos_ref, ue_ref, eye_ref, out_ref):
        g3 = g_ref[...].reshape(LB, B, 128)
        pm = par_ref[...]
        ue = ue_ref[...]
        el = eye_ref[0]
        er = eye_ref[1]
        for j in range(LB):
            gl = lax.dot_general(
                el, g3[j], (((1,), (1,)), ((), ())),
                preferred_element_type=jnp.float32)
            gr = lax.dot_general(
                er, g3[j], (((1,), (1,)), ((), ())),
                preferred_element_type=jnp.float32)
            half = gl + pm[j] * (gr - gl)
            unit = lax.dot_general(
                ue, ph_ref[:, j, :],
                (((0,), (0,)), ((), ())),
                preferred_element_type=jnp.float32,
            )
            out_ref[j] = half + unit + pos_ref[0, :, j][:, None]

    return pl.pallas_call(
        body,
        grid=(L // LB,),
        in_specs=[
            pl.BlockSpec((LB * B, 128), lambda i: (i, 0)),
            pl.BlockSpec((LB, 1, B), lambda i: (i, 0, 0)),
            pl.BlockSpec((7, LB, B), lambda i: (0, i, 0)),
            pl.BlockSpec((1, D, LB), lambda i: (i, 0, 0)),
            pl.BlockSpec((7, D), lambda i: (0, 0)),
            pl.BlockSpec((2, D, 128), lambda i: (0, 0, 0)),
        ],
        out_specs=pl.BlockSpec((LB, D, B), lambda i: (i, 0, 0)),
        out_shape=jax.ShapeDtypeStruct((L, D, B), jnp.float32),
    )(g128, par3, ph_t, pos_t, unit_embed, eyelr)


LB2 = 8


def _tc_out2t(meta_t, pad_t, case_embed):
    def body(meta_ref, pad_ref, case_ref, out_ref):
        mt = meta_ref[...]
        case = case_ref[...]
        for j in range(LB2):
            rhs = mt * pad_ref[0, :, j][:, None]
            out_ref[j] = lax.dot_general(
                case, rhs,
                (((0,), (0,)), ((), ())),
                preferred_element_type=jnp.float32,
            )

    return pl.pallas_call(
        body,
        grid=(L // LB2,),
        in_specs=[
            pl.BlockSpec((D, B), lambda i: (0, 0)),
            pl.BlockSpec((1, D, LB2), lambda i: (i, 0, 0)),
            pl.BlockSpec((D, D), lambda i: (0, 0)),
        ],
        out_specs=pl.BlockSpec((LB2, D, B), lambda i: (i, 0, 0)),
        out_shape=jax.ShapeDtypeStruct((L, D, B), jnp.float32),
    )(meta_t, pad_t, case_embed)


def kernel(sequence, meta_info, ph_dimensions, token_table, pos_table,
           case_embed, unit_embed, padding):
    table_t = jnp.transpose(token_table, (1, 0))
    eye64 = jnp.eye(D, dtype=jnp.float32)
    table2 = _tc_repack(table_t, eye64)

    seq_t = jnp.transpose(sequence, (1, 0))
    idx_t = seq_t.reshape(NROWS).astype(jnp.int32)
    high = idx_t >= H
    idx2 = jnp.where(high, idx_t - H, idx_t)
    par3 = high.astype(jnp.float32).reshape(L, 1, B)
    eyelr = jnp.stack([jnp.eye(D, 128, dtype=jnp.float32),
                       jnp.eye(D, 128, k=D, dtype=jnp.float32)])

    g128 = _sc_gather_pairs(idx2, table2)

    ph_t = jnp.transpose(ph_dimensions.astype(jnp.float32), (2, 1, 0))
    pos_t = jnp.transpose(pos_table, (1, 0))
    pos_r = jnp.transpose(pos_t.reshape(D, L // LB, LB), (1, 0, 2))
    out1t = _tc_out1t(g128, par3, ph_t, pos_r, unit_embed, eyelr)

    meta_t = jnp.transpose(meta_info, (1, 0))
    pad_t = jnp.transpose(padding, (1, 0))
    pad_r = jnp.transpose(pad_t.reshape(D, L // LB2, LB2), (1, 0, 2))
    out2t = _tc_out2t(meta_t, pad_r, case_embed)

    out1 = jnp.transpose(out1t, (2, 0, 1))
    out2 = jnp.transpose(out2t, (2, 0, 1))
    return (out1, out2)
